# baseline (speedup 1.0000x reference)
import jax
import jax.numpy as jnp
from jax import lax
from jax.experimental import pallas as pl
from jax.experimental.pallas import tpu as pltpu
from jax.experimental.pallas import tpu_sc as plsc

NUM_CORES = 2
NUM_SUBCORES = 16
NUM_WORKERS = NUM_CORES * NUM_SUBCORES

BATCH = 16384
HIDDEN = 128
IDX_CHUNK = 64
B_PER_W = BATCH // NUM_WORKERS
N_CHUNKS = B_PER_W // IDX_CHUNK


def _gather_body(table_hbm, idx_hbm, out_hbm, idx_v, rows_v, isem, gsem):
    wid = lax.axis_index("s") * NUM_CORES + lax.axis_index("c")
    base = wid * B_PER_W
    idx_copies = [
        pltpu.async_copy(
            idx_hbm.at[pl.ds(base + j * IDX_CHUNK, IDX_CHUNK)],
            idx_v.at[pl.ds(j * IDX_CHUNK, IDX_CHUNK)],
            isem,
        )
        for j in range(N_CHUNKS)
    ]
    gathers = []
    for j in range(N_CHUNKS):
        idx_copies[j].wait()
        gathers.append(
            pltpu.async_copy(
                table_hbm.at[idx_v.at[pl.ds(j * IDX_CHUNK, IDX_CHUNK)]],
                rows_v.at[pl.ds(j * IDX_CHUNK, IDX_CHUNK)],
                gsem,
            )
        )
    stores = []
    for j in range(N_CHUNKS):
        gathers[j].wait()
        stores.append(
            pltpu.async_copy(
                rows_v.at[pl.ds(j * IDX_CHUNK, IDX_CHUNK)],
                out_hbm.at[pl.ds(base + j * IDX_CHUNK, IDX_CHUNK)],
                isem,
            )
        )
    for s in stores:
        s.wait()


@jax.jit
def _embed(labels, embedding_table):
    mesh = plsc.VectorSubcoreMesh(
        core_axis_name="c", subcore_axis_name="s",
        num_cores=NUM_CORES, num_subcores=NUM_SUBCORES,
    )
    call = pl.kernel(
        _gather_body,
        out_type=jax.ShapeDtypeStruct((BATCH, HIDDEN), jnp.float32),
        mesh=mesh,
        scratch_types=[
            pltpu.VMEM((B_PER_W,), jnp.int32),
            pltpu.VMEM((B_PER_W, HIDDEN), jnp.float32),
            pltpu.SemaphoreType.DMA,
            pltpu.SemaphoreType.DMA,
        ],
    )
    return call(embedding_table, labels)


def kernel(labels, embedding_table):
    return _embed(labels.astype(jnp.int32), embedding_table)


# speedup vs baseline: 1.0252x; 1.0252x over previous
import jax
import jax.numpy as jnp
from jax import lax
from jax.experimental import pallas as pl
from jax.experimental.pallas import tpu as pltpu
from jax.experimental.pallas import tpu_sc as plsc

NUM_CORES = 2
NUM_SUBCORES = 16
NUM_WORKERS = NUM_CORES * NUM_SUBCORES

BATCH = 16384
HIDDEN = 128
IDX_CHUNK = 128
B_PER_W = BATCH // NUM_WORKERS
N_CHUNKS = B_PER_W // IDX_CHUNK


def _gather_body(table_hbm, idx_hbm, out_hbm, idx_v, rows_v, gsem):
    wid = lax.axis_index("s") * NUM_CORES + lax.axis_index("c")
    base = wid * B_PER_W
    pltpu.sync_copy(idx_hbm.at[pl.ds(base, B_PER_W)], idx_v)
    gathers = []
    for j in range(N_CHUNKS):
        gathers.append(
            pltpu.async_copy(
                table_hbm.at[idx_v.at[pl.ds(j * IDX_CHUNK, IDX_CHUNK)]],
                rows_v.at[pl.ds(j * IDX_CHUNK, IDX_CHUNK)],
                gsem,
            )
        )
    for g in gathers:
        g.wait()
    pltpu.sync_copy(rows_v, out_hbm.at[pl.ds(base, B_PER_W)])


@jax.jit
def _embed(labels, embedding_table):
    mesh = plsc.VectorSubcoreMesh(
        core_axis_name="c", subcore_axis_name="s",
        num_cores=NUM_CORES, num_subcores=NUM_SUBCORES,
    )
    call = pl.kernel(
        _gather_body,
        out_type=jax.ShapeDtypeStruct((BATCH, HIDDEN), jnp.float32),
        mesh=mesh,
        scratch_types=[
            pltpu.VMEM((B_PER_W,), jnp.int32),
            pltpu.VMEM((B_PER_W, HIDDEN), jnp.float32),
            pltpu.SemaphoreType.DMA,
        ],
    )
    return call(embedding_table, labels)


def kernel(labels, embedding_table):
    return _embed(labels.astype(jnp.int32), embedding_table)
